# baseline (speedup 1.0000x reference)
import functools

import jax
import jax.numpy as jnp
from jax import lax
from jax.experimental import pallas as pl
from jax.experimental.pallas import tpu as pltpu
from jax.experimental.pallas import tpu_sc as plsc

M = 8192
HIDDEN = 2048
INTER = 1024
E = 16
TOPK = 2
R = M * TOPK
BR = 256
NB = R // BR + E
P = NB * BR
CLIP_LO, CLIP_HI = -10.0, 10.0
CLIP_GATE = 10.0

NW = 32
ROWS_W = P // NW
GCH = 8
NCH = ROWS_W // GCH
TOK_W = M // NW
TCH = TOK_W // GCH


NBUF = 4


def _sc_gather_rows(table, idx, n_out):
    nch = idx.shape[1]
    rows_w = nch * GCH
    mesh = plsc.VectorSubcoreMesh(core_axis_name="c", subcore_axis_name="s")

    @functools.partial(
        pl.kernel,
        out_type=jax.ShapeDtypeStruct((n_out, HIDDEN), jnp.float32),
        mesh=mesh,
        scratch_types=[
            pltpu.VMEM((nch, GCH), jnp.int32),
            pltpu.VMEM((NBUF, GCH, HIDDEN), jnp.float32),
        ] + [pltpu.SemaphoreType.DMA] * (2 * NBUF),
    )
    def k(table_hbm, idx_hbm, out_hbm, idx_v, buf, *sems):
        gsem, wsem = sems[:NBUF], sems[NBUF:]
        wid = lax.axis_index("s") * 2 + lax.axis_index("c")
        pltpu.sync_copy(idx_hbm.at[wid], idx_v)
        base = wid * rows_w

        def group(g, carry):
            c0 = g * NBUF
            gcps = [
                pltpu.async_copy(
                    table_hbm.at[idx_v.at[c0 + b]], buf.at[b], gsem[b])
                for b in range(NBUF)
            ]
            wcps = []
            for b in range(NBUF):
                gcps[b].wait()
                wcps.append(pltpu.async_copy(
                    buf.at[b],
                    out_hbm.at[pl.ds(base + (c0 + b) * GCH, GCH)],
                    wsem[b]))
            for w in wcps:
                w.wait()
            return carry

        lax.fori_loop(0, nch // NBUF, group, 0)

    return k(table, idx)


def _sc_combine_gather(y, idx2):
    nch = 2 * TCH
    mesh = plsc.VectorSubcoreMesh(core_axis_name="c", subcore_axis_name="s")

    @functools.partial(
        pl.kernel,
        out_type=jax.ShapeDtypeStruct((2 * M, HIDDEN), jnp.float32),
        mesh=mesh,
        scratch_types=[
            pltpu.VMEM((nch, GCH), jnp.int32),
            pltpu.VMEM((NBUF, GCH, HIDDEN), jnp.float32),
        ] + [pltpu.SemaphoreType.DMA] * (2 * NBUF),
    )
    def k(y_hbm, idx_hbm, out_hbm, idx_v, buf, *sems):
        gsem, wsem = sems[:NBUF], sems[NBUF:]
        wid = lax.axis_index("s") * 2 + lax.axis_index("c")
        pltpu.sync_copy(idx_hbm.at[wid], idx_v)
        base = wid * TOK_W

        def group(g, carry):
            c0 = g * NBUF
            gcps = [
                pltpu.async_copy(y_hbm.at[idx_v.at[c0 + b]], buf.at[b], gsem[b])
                for b in range(NBUF)
            ]
            wcps = []
            for b in range(NBUF):
                c = c0 + b
                off = base + c * GCH + jnp.where(c >= TCH, M - TCH * GCH, 0)
                gcps[b].wait()
                wcps.append(pltpu.async_copy(
                    buf.at[b], out_hbm.at[pl.ds(off, GCH)], wsem[b]))
            for w in wcps:
                w.wait()
            return carry

        lax.fori_loop(0, nch // NBUF, group, 0)

    return k(y, idx2)


def _tc_grouped_gemm(x, gw, uw, dw, gate_col, be_ext):
    def body(be_ref, x_ref, gw_ref, uw_ref, dw_ref, g_ref, y_ref):
        @pl.when(pl.program_id(0) < be_ref[NB])
        def _():
            x = x_ref[...].astype(jnp.bfloat16)
            dn = (((1,), (1,)), ((), ()))
            go = lax.dot_general(x, gw_ref[0], dn, preferred_element_type=jnp.float32)
            uo = lax.dot_general(x, uw_ref[0], dn, preferred_element_type=jnp.float32)
            go = jnp.minimum(go, CLIP_GATE)
            uo = jnp.clip(uo, CLIP_LO, CLIP_HI)
            h = (go * jax.nn.sigmoid(go)) * uo * g_ref[...]
            y_ref[...] = lax.dot_general(
                h.astype(jnp.bfloat16), dw_ref[0], dn,
                preferred_element_type=jnp.float32)

    def _row(i, be):
        return (jnp.minimum(i, be[NB] - 1), 0)

    grid_spec = pltpu.PrefetchScalarGridSpec(
        num_scalar_prefetch=1,
        grid=(NB,),
        in_specs=[
            pl.BlockSpec((BR, HIDDEN), _row),
            pl.BlockSpec((1, INTER, HIDDEN), lambda i, be: (be[jnp.minimum(i, be[NB] - 1)], 0, 0)),
            pl.BlockSpec((1, INTER, HIDDEN), lambda i, be: (be[jnp.minimum(i, be[NB] - 1)], 0, 0)),
            pl.BlockSpec((1, HIDDEN, INTER), lambda i, be: (be[jnp.minimum(i, be[NB] - 1)], 0, 0)),
            pl.BlockSpec((BR, 1), _row),
        ],
        out_specs=pl.BlockSpec((BR, HIDDEN), _row),
    )
    return pl.pallas_call(
        body,
        grid_spec=grid_spec,
        out_shape=jax.ShapeDtypeStruct((P, HIDDEN), jnp.float32),
        compiler_params=pltpu.CompilerParams(dimension_semantics=("arbitrary",)),
    )(be_ext, x, gw, uw, dw, gate_col)


def _tc_cast_bf16(w):
    blk = (1,) + w.shape[1:]

    def body(w_ref, o_ref):
        o_ref[...] = w_ref[...].astype(jnp.bfloat16)

    return pl.pallas_call(
        body,
        grid=(w.shape[0],),
        in_specs=[pl.BlockSpec(blk, lambda i: (i, 0, 0))],
        out_specs=pl.BlockSpec(blk, lambda i: (i, 0, 0)),
        out_shape=jax.ShapeDtypeStruct(w.shape, jnp.bfloat16),
    )(w)


def _tc_pair_add(ab):
    blk = 512

    def body(a_ref, b_ref, o_ref):
        o_ref[...] = a_ref[...] + b_ref[...]

    return pl.pallas_call(
        body,
        grid=(M // blk,),
        in_specs=[
            pl.BlockSpec((blk, HIDDEN), lambda i: (i, 0)),
            pl.BlockSpec((blk, HIDDEN), lambda i: (i + M // blk, 0)),
        ],
        out_specs=pl.BlockSpec((blk, HIDDEN), lambda i: (i, 0)),
        out_shape=jax.ShapeDtypeStruct((M, HIDDEN), jnp.float32),
    )(ab, ab)


def _dispatch_plan(flat_idx, flat_gate):
    i32 = jnp.int32
    expert_id = flat_idx.reshape(-1).astype(i32)
    gate = flat_gate.reshape(-1)
    perm = jnp.argsort(expert_id, stable=True).astype(i32)
    e_s = expert_id[perm]
    e_range = jnp.arange(E, dtype=i32)
    counts = jnp.sum(e_range[:, None] == expert_id[None, :], axis=1, dtype=i32)
    offsets = jnp.cumsum(counts) - counts
    blocks_per_e = (counts + BR - 1) // BR
    cum_blocks = jnp.cumsum(blocks_per_e)
    pad_offset = (cum_blocks - blocks_per_e) * BR
    s_ar = jnp.arange(R, dtype=i32)
    q_of_s = pad_offset[e_s] + (s_ar - offsets[e_s])
    q_ar = jnp.arange(P, dtype=i32)
    e_q = jnp.sum((cum_blocks * BR)[:, None] <= q_ar[None, :], axis=0,
                  dtype=i32)
    e_qc = jnp.minimum(e_q, E - 1)
    d = q_ar - pad_offset[e_qc]
    valid = (d < counts[e_qc]) & (e_q < E)
    s_q = jnp.clip(offsets[e_qc] + jnp.minimum(d, counts[e_qc] - 1), 0, R - 1)
    rowid_q = perm[s_q]
    src_token = jnp.where(valid, rowid_q // TOPK, q_ar % M)
    gate_padded = jnp.where(valid, gate[rowid_q], 0.0)
    i_ar = jnp.arange(NB, dtype=i32)
    be = jnp.minimum(
        jnp.sum(cum_blocks[:, None] <= i_ar[None, :], axis=0, dtype=i32),
        E - 1)
    be_ext = jnp.concatenate([be, cum_blocks[-1:].astype(i32)])
    inv_perm = jnp.argsort(perm).astype(i32)
    inv_padded = q_of_s[inv_perm]
    idx_a = inv_padded[0::2].reshape(NW, TCH, GCH)
    idx_b = inv_padded[1::2].reshape(NW, TCH, GCH)
    idx2 = jnp.concatenate([idx_a, idx_b], axis=1)
    return src_token.reshape(NW, NCH, GCH), gate_padded.reshape(P, 1), be_ext, idx2


def kernel(flat_h, flat_idx, flat_gate, gate_weight, up_weight, down_weight):
    src_token, gate_padded, be_ext, idx2 = _dispatch_plan(flat_idx, flat_gate)
    gathered = _sc_gather_rows(flat_h, src_token, P)
    y = _tc_grouped_gemm(gathered,
                         _tc_cast_bf16(gate_weight),
                         _tc_cast_bf16(up_weight),
                         _tc_cast_bf16(down_weight),
                         gate_padded, be_ext)
    ab = _sc_combine_gather(y, idx2)
    return _tc_pair_add(ab)


# speedup vs baseline: 1.0907x; 1.0907x over previous
import functools

import jax
import jax.numpy as jnp
from jax import lax
from jax.experimental import pallas as pl
from jax.experimental.pallas import tpu as pltpu
from jax.experimental.pallas import tpu_sc as plsc

M = 8192
HIDDEN = 2048
INTER = 1024
E = 16
TOPK = 2
R = M * TOPK
BR = 256
NB = R // BR + E
P = NB * BR
CLIP_LO, CLIP_HI = -10.0, 10.0
CLIP_GATE = 10.0

NW = 32
ROWS_W = P // NW
GCH = 8
NCH = ROWS_W // GCH
TOK_W = M // NW
TCH = TOK_W // GCH


NBUF = 4


def _sc_gather_rows(table, idx, n_out):
    nch = idx.shape[1]
    rows_w = nch * GCH
    mesh = plsc.VectorSubcoreMesh(core_axis_name="c", subcore_axis_name="s")

    @functools.partial(
        pl.kernel,
        out_type=jax.ShapeDtypeStruct((n_out, HIDDEN), jnp.float32),
        mesh=mesh,
        scratch_types=[
            pltpu.VMEM((nch, GCH), jnp.int32),
            pltpu.VMEM((NBUF, GCH, HIDDEN), jnp.float32),
        ] + [pltpu.SemaphoreType.DMA] * (2 * NBUF),
    )
    def k(table_hbm, idx_hbm, out_hbm, idx_v, buf, *sems):
        gsem, wsem = sems[:NBUF], sems[NBUF:]
        wid = lax.axis_index("s") * 2 + lax.axis_index("c")
        pltpu.sync_copy(idx_hbm.at[wid], idx_v)
        base = wid * rows_w

        def group(g, carry):
            c0 = g * NBUF
            gcps = [
                pltpu.async_copy(
                    table_hbm.at[idx_v.at[c0 + b]], buf.at[b], gsem[b])
                for b in range(NBUF)
            ]
            wcps = []
            for b in range(NBUF):
                gcps[b].wait()
                wcps.append(pltpu.async_copy(
                    buf.at[b],
                    out_hbm.at[pl.ds(base + (c0 + b) * GCH, GCH)],
                    wsem[b]))
            for w in wcps:
                w.wait()
            return carry

        lax.fori_loop(0, nch // NBUF, group, 0)

    return k(table, idx)


def _sc_combine_gather(y, idx2):
    nch = 2 * TCH
    mesh = plsc.VectorSubcoreMesh(core_axis_name="c", subcore_axis_name="s")

    @functools.partial(
        pl.kernel,
        out_type=jax.ShapeDtypeStruct((2 * M, HIDDEN), jnp.float32),
        mesh=mesh,
        scratch_types=[
            pltpu.VMEM((nch, GCH), jnp.int32),
            pltpu.VMEM((NBUF, GCH, HIDDEN), jnp.float32),
        ] + [pltpu.SemaphoreType.DMA] * (2 * NBUF),
    )
    def k(y_hbm, idx_hbm, out_hbm, idx_v, buf, *sems):
        gsem, wsem = sems[:NBUF], sems[NBUF:]
        wid = lax.axis_index("s") * 2 + lax.axis_index("c")
        pltpu.sync_copy(idx_hbm.at[wid], idx_v)
        base = wid * TOK_W

        def group(g, carry):
            c0 = g * NBUF
            gcps = [
                pltpu.async_copy(y_hbm.at[idx_v.at[c0 + b]], buf.at[b], gsem[b])
                for b in range(NBUF)
            ]
            wcps = []
            for b in range(NBUF):
                c = c0 + b
                off = base + c * GCH + jnp.where(c >= TCH, M - TCH * GCH, 0)
                gcps[b].wait()
                wcps.append(pltpu.async_copy(
                    buf.at[b], out_hbm.at[pl.ds(off, GCH)], wsem[b]))
            for w in wcps:
                w.wait()
            return carry

        lax.fori_loop(0, nch // NBUF, group, 0)

    return k(y, idx2)


def _tc_grouped_gemm(x, gw, uw, dw, gate_col, be_ext):
    def body(be_ref, x_ref, gw_ref, uw_ref, dw_ref, g_ref, y_ref):
        @pl.when(pl.program_id(0) < be_ref[NB])
        def _():
            x = x_ref[...].astype(jnp.bfloat16)
            dn = (((1,), (1,)), ((), ()))
            go = lax.dot_general(x, gw_ref[0], dn, preferred_element_type=jnp.float32)
            uo = lax.dot_general(x, uw_ref[0], dn, preferred_element_type=jnp.float32)
            go = jnp.minimum(go, CLIP_GATE)
            uo = jnp.clip(uo, CLIP_LO, CLIP_HI)
            h = (go * jax.nn.sigmoid(go)) * uo * g_ref[...]
            y_ref[...] = lax.dot_general(
                h.astype(jnp.bfloat16), dw_ref[0], dn,
                preferred_element_type=jnp.float32)

    def _row(i, be):
        return (jnp.minimum(i, be[NB] - 1), 0)

    grid_spec = pltpu.PrefetchScalarGridSpec(
        num_scalar_prefetch=1,
        grid=(NB,),
        in_specs=[
            pl.BlockSpec((BR, HIDDEN), _row),
            pl.BlockSpec((1, INTER, HIDDEN), lambda i, be: (be[jnp.minimum(i, be[NB] - 1)], 0, 0)),
            pl.BlockSpec((1, INTER, HIDDEN), lambda i, be: (be[jnp.minimum(i, be[NB] - 1)], 0, 0)),
            pl.BlockSpec((1, HIDDEN, INTER), lambda i, be: (be[jnp.minimum(i, be[NB] - 1)], 0, 0)),
            pl.BlockSpec((BR, 1), _row),
        ],
        out_specs=pl.BlockSpec((BR, HIDDEN), _row),
    )
    return pl.pallas_call(
        body,
        grid_spec=grid_spec,
        out_shape=jax.ShapeDtypeStruct((P, HIDDEN), jnp.float32),
        compiler_params=pltpu.CompilerParams(dimension_semantics=("arbitrary",)),
    )(be_ext, x, gw, uw, dw, gate_col)


def _tc_counts(expert_id):
    def body(x_ref, o_ref):
        x = x_ref[...]
        lane = lax.broadcasted_iota(jnp.int32, (8, 128), 1)
        acc = jnp.zeros((8, 128), jnp.int32)
        for e in range(E):
            s = jnp.sum((x == e).astype(jnp.int32))
            acc = acc + jnp.where(lane == e, s, 0)
        o_ref[...] = acc

    out = pl.pallas_call(
        body,
        out_shape=jax.ShapeDtypeStruct((8, 128), jnp.int32),
    )(expert_id.reshape(R // 128, 128))
    return out[0, :E]


def _tc_cast_bf16(w):
    blk = (1,) + w.shape[1:]

    def body(w_ref, o_ref):
        o_ref[...] = w_ref[...].astype(jnp.bfloat16)

    return pl.pallas_call(
        body,
        grid=(w.shape[0],),
        in_specs=[pl.BlockSpec(blk, lambda i: (i, 0, 0))],
        out_specs=pl.BlockSpec(blk, lambda i: (i, 0, 0)),
        out_shape=jax.ShapeDtypeStruct(w.shape, jnp.bfloat16),
    )(w)


def _tc_pair_add(ab):
    blk = 512

    def body(a_ref, b_ref, o_ref):
        o_ref[...] = a_ref[...] + b_ref[...]

    return pl.pallas_call(
        body,
        grid=(M // blk,),
        in_specs=[
            pl.BlockSpec((blk, HIDDEN), lambda i: (i, 0)),
            pl.BlockSpec((blk, HIDDEN), lambda i: (i + M // blk, 0)),
        ],
        out_specs=pl.BlockSpec((blk, HIDDEN), lambda i: (i, 0)),
        out_shape=jax.ShapeDtypeStruct((M, HIDDEN), jnp.float32),
    )(ab, ab)


def _dispatch_plan(flat_idx, flat_gate):
    i32 = jnp.int32
    expert_id = flat_idx.reshape(-1).astype(i32)
    gate = flat_gate.reshape(-1)
    perm = jnp.argsort(expert_id, stable=True).astype(i32)
    counts = _tc_counts(expert_id)
    offsets = jnp.cumsum(counts) - counts
    blocks_per_e = (counts + BR - 1) // BR
    cum_blocks = jnp.cumsum(blocks_per_e)
    pad_offset = (cum_blocks - blocks_per_e) * BR
    i_ar = jnp.arange(NB, dtype=i32)
    be = jnp.minimum(
        jnp.sum(cum_blocks[:, None] <= i_ar[None, :], axis=0, dtype=i32),
        E - 1)
    be_ext = jnp.concatenate([be, cum_blocks[-1:].astype(i32)])
    q_ar = jnp.arange(P, dtype=i32)
    d = q_ar - jnp.repeat(pad_offset[be], BR)
    cnt_q = jnp.repeat(counts[be], BR)
    valid = (d < cnt_q) & jnp.repeat(i_ar < cum_blocks[-1], BR)
    s_q = jnp.clip(jnp.repeat(offsets[be], BR) + jnp.minimum(d, cnt_q - 1),
                   0, R - 1)
    rowid_q = perm[s_q]
    src_token = jnp.where(valid, rowid_q // TOPK, q_ar % M)
    gate_padded = jnp.where(valid, gate[rowid_q], 0.0)
    inv_perm = jnp.argsort(perm).astype(i32)
    inv_padded = (pad_offset - offsets)[expert_id] + inv_perm
    idx_a = inv_padded[0::2].reshape(NW, TCH, GCH)
    idx_b = inv_padded[1::2].reshape(NW, TCH, GCH)
    idx2 = jnp.concatenate([idx_a, idx_b], axis=1)
    return src_token.reshape(NW, NCH, GCH), gate_padded.reshape(P, 1), be_ext, idx2


def kernel(flat_h, flat_idx, flat_gate, gate_weight, up_weight, down_weight):
    src_token, gate_padded, be_ext, idx2 = _dispatch_plan(flat_idx, flat_gate)
    gathered = _sc_gather_rows(flat_h, src_token, P)
    y = _tc_grouped_gemm(gathered,
                         _tc_cast_bf16(gate_weight),
                         _tc_cast_bf16(up_weight),
                         _tc_cast_bf16(down_weight),
                         gate_padded, be_ext)
    ab = _sc_combine_gather(y, idx2)
    return _tc_pair_add(ab)


# speedup vs baseline: 1.0942x; 1.0032x over previous
import functools

import jax
import jax.numpy as jnp
from jax import lax
from jax.experimental import pallas as pl
from jax.experimental.pallas import tpu as pltpu
from jax.experimental.pallas import tpu_sc as plsc

M = 8192
HIDDEN = 2048
INTER = 1024
E = 16
TOPK = 2
R = M * TOPK
BR = 256
NB = R // BR + E
P = NB * BR
CLIP_LO, CLIP_HI = -10.0, 10.0
CLIP_GATE = 10.0

NW = 32
ROWS_W = P // NW
GCH = 8
NCH = ROWS_W // GCH
TOK_W = M // NW
TCH = TOK_W // GCH


NBUF = 4


def _sc_gather_rows(table, idx, n_out, gch=GCH, nbuf=NBUF):
    nch = idx.shape[1]
    rows_w = nch * gch
    mesh = plsc.VectorSubcoreMesh(core_axis_name="c", subcore_axis_name="s")

    @functools.partial(
        pl.kernel,
        out_type=jax.ShapeDtypeStruct((n_out, HIDDEN), jnp.float32),
        mesh=mesh,
        scratch_types=[
            pltpu.VMEM((nch, gch), jnp.int32),
            pltpu.VMEM((nbuf, gch, HIDDEN), jnp.float32),
        ] + [pltpu.SemaphoreType.DMA] * (2 * nbuf),
    )
    def k(table_hbm, idx_hbm, out_hbm, idx_v, buf, *sems):
        gsem, wsem = sems[:nbuf], sems[nbuf:]
        wid = lax.axis_index("s") * 2 + lax.axis_index("c")
        pltpu.sync_copy(idx_hbm.at[wid], idx_v)
        base = wid * rows_w

        def group(g, carry):
            c0 = g * nbuf
            gcps = [
                pltpu.async_copy(
                    table_hbm.at[idx_v.at[c0 + b]], buf.at[b], gsem[b])
                for b in range(nbuf)
            ]
            wcps = []
            for b in range(nbuf):
                gcps[b].wait()
                wcps.append(pltpu.async_copy(
                    buf.at[b],
                    out_hbm.at[pl.ds(base + (c0 + b) * gch, gch)],
                    wsem[b]))
            for w in wcps:
                w.wait()
            return carry

        lax.fori_loop(0, nch // nbuf, group, 0)

    return k(table, idx)


def _sc_combine_gather(y, idx2):
    nch = 2 * TCH
    mesh = plsc.VectorSubcoreMesh(core_axis_name="c", subcore_axis_name="s")

    @functools.partial(
        pl.kernel,
        out_type=jax.ShapeDtypeStruct((2 * M, HIDDEN), jnp.float32),
        mesh=mesh,
        scratch_types=[
            pltpu.VMEM((nch, GCH), jnp.int32),
            pltpu.VMEM((NBUF, GCH, HIDDEN), jnp.float32),
        ] + [pltpu.SemaphoreType.DMA] * (2 * NBUF),
    )
    def k(y_hbm, idx_hbm, out_hbm, idx_v, buf, *sems):
        gsem, wsem = sems[:NBUF], sems[NBUF:]
        wid = lax.axis_index("s") * 2 + lax.axis_index("c")
        pltpu.sync_copy(idx_hbm.at[wid], idx_v)
        base = wid * TOK_W

        def group(g, carry):
            c0 = g * NBUF
            gcps = [
                pltpu.async_copy(y_hbm.at[idx_v.at[c0 + b]], buf.at[b], gsem[b])
                for b in range(NBUF)
            ]
            wcps = []
            for b in range(NBUF):
                c = c0 + b
                off = base + c * GCH + jnp.where(c >= TCH, M - TCH * GCH, 0)
                gcps[b].wait()
                wcps.append(pltpu.async_copy(
                    buf.at[b], out_hbm.at[pl.ds(off, GCH)], wsem[b]))
            for w in wcps:
                w.wait()
            return carry

        lax.fori_loop(0, nch // NBUF, group, 0)

    return k(y, idx2)


def _tc_grouped_gemm(x, gw, uw, dw, gate_col, be_ext):
    def body(be_ref, x_ref, gw_ref, uw_ref, dw_ref, g_ref, y_ref):
        @pl.when(pl.program_id(0) < be_ref[NB])
        def _():
            x = x_ref[...].astype(jnp.bfloat16)
            dn = (((1,), (1,)), ((), ()))
            go = lax.dot_general(x, gw_ref[0], dn, preferred_element_type=jnp.float32)
            uo = lax.dot_general(x, uw_ref[0], dn, preferred_element_type=jnp.float32)
            go = jnp.minimum(go, CLIP_GATE)
            uo = jnp.clip(uo, CLIP_LO, CLIP_HI)
            h = (go * jax.nn.sigmoid(go)) * uo * g_ref[...]
            y_ref[...] = lax.dot_general(
                h.astype(jnp.bfloat16), dw_ref[0], dn,
                preferred_element_type=jnp.float32)

    def _row(i, be):
        return (jnp.minimum(i, be[NB] - 1), 0)

    grid_spec = pltpu.PrefetchScalarGridSpec(
        num_scalar_prefetch=1,
        grid=(NB,),
        in_specs=[
            pl.BlockSpec((BR, HIDDEN), _row),
            pl.BlockSpec((1, INTER, HIDDEN), lambda i, be: (be[jnp.minimum(i, be[NB] - 1)], 0, 0)),
            pl.BlockSpec((1, INTER, HIDDEN), lambda i, be: (be[jnp.minimum(i, be[NB] - 1)], 0, 0)),
            pl.BlockSpec((1, HIDDEN, INTER), lambda i, be: (be[jnp.minimum(i, be[NB] - 1)], 0, 0)),
            pl.BlockSpec((BR, 1), _row),
        ],
        out_specs=pl.BlockSpec((BR, HIDDEN), _row),
    )
    return pl.pallas_call(
        body,
        grid_spec=grid_spec,
        out_shape=jax.ShapeDtypeStruct((P, HIDDEN), jnp.float32),
        compiler_params=pltpu.CompilerParams(dimension_semantics=("arbitrary",)),
    )(be_ext, x, gw, uw, dw, gate_col)


def _tc_counts(expert_id):
    def body(x_ref, o_ref):
        x = x_ref[...]
        lane = lax.broadcasted_iota(jnp.int32, (8, 128), 1)
        acc = jnp.zeros((8, 128), jnp.int32)
        for e in range(E):
            s = jnp.sum((x == e).astype(jnp.int32))
            acc = acc + jnp.where(lane == e, s, 0)
        o_ref[...] = acc

    out = pl.pallas_call(
        body,
        out_shape=jax.ShapeDtypeStruct((8, 128), jnp.int32),
    )(expert_id.reshape(R // 128, 128))
    return out[0, :E]


def _tc_cast_bf16(w):
    blk = (1,) + w.shape[1:]

    def body(w_ref, o_ref):
        o_ref[...] = w_ref[...].astype(jnp.bfloat16)

    return pl.pallas_call(
        body,
        grid=(w.shape[0],),
        in_specs=[pl.BlockSpec(blk, lambda i: (i, 0, 0))],
        out_specs=pl.BlockSpec(blk, lambda i: (i, 0, 0)),
        out_shape=jax.ShapeDtypeStruct(w.shape, jnp.bfloat16),
    )(w)


def _tc_pair_add(ab):
    blk = 512

    def body(a_ref, b_ref, o_ref):
        o_ref[...] = a_ref[...] + b_ref[...]

    return pl.pallas_call(
        body,
        grid=(M // blk,),
        in_specs=[
            pl.BlockSpec((blk, HIDDEN), lambda i: (i, 0)),
            pl.BlockSpec((blk, HIDDEN), lambda i: (i + M // blk, 0)),
        ],
        out_specs=pl.BlockSpec((blk, HIDDEN), lambda i: (i, 0)),
        out_shape=jax.ShapeDtypeStruct((M, HIDDEN), jnp.float32),
    )(ab, ab)


def _dispatch_plan(flat_idx, flat_gate):
    i32 = jnp.int32
    expert_id = flat_idx.reshape(-1).astype(i32)
    gate = flat_gate.reshape(-1)
    perm = jnp.argsort(expert_id, stable=True).astype(i32)
    counts = _tc_counts(expert_id)
    offsets = jnp.cumsum(counts) - counts
    blocks_per_e = (counts + BR - 1) // BR
    cum_blocks = jnp.cumsum(blocks_per_e)
    pad_offset = (cum_blocks - blocks_per_e) * BR
    i_ar = jnp.arange(NB, dtype=i32)
    be = jnp.minimum(
        jnp.sum(cum_blocks[:, None] <= i_ar[None, :], axis=0, dtype=i32),
        E - 1)
    be_ext = jnp.concatenate([be, cum_blocks[-1:].astype(i32)])
    q_ar = jnp.arange(P, dtype=i32)
    d = q_ar - jnp.repeat(pad_offset[be], BR)
    cnt_q = jnp.repeat(counts[be], BR)
    valid = (d < cnt_q) & jnp.repeat(i_ar < cum_blocks[-1], BR)
    s_q = jnp.clip(jnp.repeat(offsets[be], BR) + jnp.minimum(d, cnt_q - 1),
                   0, R - 1)
    rowid_q = perm[s_q]
    src_token = jnp.where(valid, rowid_q // TOPK, q_ar % M)
    gate_padded = jnp.where(valid, gate[rowid_q], 0.0)
    inv_perm = jnp.argsort(perm).astype(i32)
    inv_padded = (pad_offset - offsets)[expert_id] + inv_perm
    idx_a = inv_padded[0::2].reshape(NW, TCH, GCH)
    idx_b = inv_padded[1::2].reshape(NW, TCH, GCH)
    idx2 = jnp.concatenate([idx_a, idx_b], axis=1)
    return src_token.reshape(NW, NCH, GCH), gate_padded.reshape(P, 1), be_ext, idx2


def kernel(flat_h, flat_idx, flat_gate, gate_weight, up_weight, down_weight):
    src_token, gate_padded, be_ext, idx2 = _dispatch_plan(flat_idx, flat_gate)
    gathered = _sc_gather_rows(
        flat_h, src_token.reshape(NW, ROWS_W // 16, 16), P, gch=16, nbuf=2)
    y = _tc_grouped_gemm(gathered,
                         _tc_cast_bf16(gate_weight),
                         _tc_cast_bf16(up_weight),
                         _tc_cast_bf16(down_weight),
                         gate_padded, be_ext)
    ab = _sc_combine_gather(y, idx2)
    return _tc_pair_add(ab)
